# baseline (speedup 1.0000x reference)
import functools

import jax
import jax.numpy as jnp
from jax import lax
from jax.experimental import pallas as pl
from jax.experimental.pallas import tpu as pltpu
from jax.experimental.pallas import tpu_sc as plsc

B = 4096
DU = 768
DA = 256
HEADS = 8
NW = 32
ROWS_W = B // NW
CH = 16
NCH = ROWS_W // CH


def _sc_body(xt_hbm, rew_hbm, subt_hbm, act_hbm, out_hbm,
             xtbuf, mbuf, actrep, rew_v, sub_v):
    wid = lax.axis_index("c") * 16 + lax.axis_index("s")
    base = wid * ROWS_W

    pltpu.sync_copy(rew_hbm.at[pl.ds(base, ROWS_W)], rew_v)
    pltpu.sync_copy(subt_hbm.at[:, pl.ds(base, ROWS_W)], sub_v)

    def rep_body(i, _):
        h = lax.div(i, CH)
        r = lax.rem(i, CH)
        pltpu.sync_copy(act_hbm.at[h], actrep.at[h, r])
        return 0
    lax.fori_loop(0, HEADS * CH, rep_body, 0)

    def step(i, _):
        c = lax.div(i, HEADS)
        h = lax.rem(i, HEADS)
        off = c * CH
        row0 = base + off

        @pl.when(h == 0)
        def _stage():
            pltpu.sync_copy(xt_hbm.at[pl.ds(row0, CH)], xtbuf)

        rew16 = rew_v[pl.ds(off, CH)]
        sub16 = sub_v[h, pl.ds(off, CH)]
        m = (rew16 == 1) & (sub16 >= 0.1)
        mf = jnp.where(m, 1.0, 0.0).astype(jnp.float32)

        def fast():
            pltpu.sync_copy(
                xtbuf, out_hbm.at[h, pl.ds(row0, CH), pl.ds(0, DU)])

        def slow():
            def rowfn(r, _):
                mr = jnp.max(
                    jnp.where(lax.iota(jnp.int32, CH) == r, mf, 0.0))

                def vecfn(v, _):
                    sl = pl.ds(v * 16, 16)
                    mbuf[r, sl] = xtbuf[r, sl] * mr
                    return 0
                lax.fori_loop(0, DU // 16, vecfn, 0)
                return 0
            lax.fori_loop(0, CH, rowfn, 0)
            pltpu.sync_copy(
                mbuf, out_hbm.at[h, pl.ds(row0, CH), pl.ds(0, DU)])

        lax.cond(jnp.sum(mf) >= CH - 0.5, fast, slow)
        pltpu.sync_copy(
            actrep.at[h], out_hbm.at[h, pl.ds(row0, CH), pl.ds(DU, DA)])
        return 0

    lax.fori_loop(0, NCH * HEADS, step, 0)


_sc_call = functools.partial(
    pl.kernel,
    out_type=jax.ShapeDtypeStruct((HEADS, B, DU + DA), jnp.float32),
    mesh=plsc.VectorSubcoreMesh(core_axis_name="c", subcore_axis_name="s"),
    compiler_params=pltpu.CompilerParams(needs_layout_passes=False),
    scratch_types=[
        pltpu.VMEM((CH, DU), jnp.float32),
        pltpu.VMEM((CH, DU), jnp.float32),
        pltpu.VMEM((HEADS, CH, DA), jnp.float32),
        pltpu.VMEM((ROWS_W,), jnp.int32),
        pltpu.VMEM((HEADS, ROWS_W), jnp.float32),
    ],
)(_sc_body)


def kernel(xt, rewards, subset, action):
    xt2 = xt.reshape(B, DU)
    subt = subset.T
    out = _sc_call(xt2, rewards, subt, action)
    return out.reshape(HEADS, B, 1, DU + DA)
